# Initial kernel scaffold; baseline (speedup 1.0000x reference)
import jax
import jax.numpy as jnp
from jax.experimental import pallas as pl


def kernel(x, edge_index, edge_weight, W):
    raise NotImplementedError("write your pallas kernel here")


# speedup vs baseline: 3.0750x; 3.0750x over previous
import functools

import jax
import jax.numpy as jnp
from jax import lax
from jax.experimental import pallas as pl
from jax.experimental.pallas import tpu as pltpu
from jax.experimental.pallas import tpu_sc as plsc

_NC = 2
_NS = 16
_LANES = 16
_CHUNK = 80


def _spmm(xr, meta, wr, n_nodes):
    _, dh = xr.shape
    n = n_nodes
    nch = meta.shape[1]
    rpt = (n // _NS) // 8 * 8
    rem = n - _NS * rpt

    mesh = plsc.VectorSubcoreMesh(
        core_axis_name="c", subcore_axis_name="s", num_cores=_NC, num_subcores=_NS
    )

    @functools.partial(
        pl.kernel,
        mesh=mesh,
        out_type=jax.ShapeDtypeStruct((_NC, n, dh), jnp.float32),
        scratch_types=[
            pltpu.VMEM((2, _CHUNK), jnp.int32),
            pltpu.VMEM((1, _CHUNK), jnp.float32),
            pltpu.VMEM((_CHUNK,), jnp.int32),
            pltpu.VMEM((_CHUNK, dh), jnp.float32),
            pltpu.VMEM_SHARED((n, dh), jnp.float32),
            pltpu.SemaphoreType.DMA,
        ],
    )
    def k(xr_hbm, meta_hbm, w_hbm, out_hbm, mb, wb, gb, rows, agg, sem):
        c = lax.axis_index("c")
        s = lax.axis_index("s")
        rbase = pl.multiple_of(s * rpt, 8)
        tail = _NS * rpt
        def zrow(r, carry):
            for v in range(dh // _LANES):
                rows[r, pl.ds(v * _LANES, _LANES)] = jnp.zeros((_LANES,), jnp.float32)
            return carry
        lax.fori_loop(0, _CHUNK, zrow, None)
        nz_full = rpt // _CHUNK
        for kz in range(nz_full):
            pltpu.sync_copy(rows, agg.at[pl.ds(rbase + kz * _CHUNK, _CHUNK)])
        zrem = rpt - nz_full * _CHUNK
        if zrem:
            pltpu.sync_copy(
                rows.at[pl.ds(0, zrem)],
                agg.at[pl.ds(rbase + nz_full * _CHUNK, zrem)],
            )
        if rem:
            @pl.when(s == _NS - 1)
            def _zero_tail():
                pltpu.sync_copy(rows.at[pl.ds(0, rem)], agg.at[pl.ds(tail, rem)])
        plsc.subcore_barrier()

        def chunk(j, carry):
            pltpu.sync_copy(meta_hbm.at[s, j], mb)
            pltpu.sync_copy(w_hbm.at[s, j], wb)
            for v in range(_CHUNK // _LANES):
                sl = pl.ds(v * _LANES, _LANES)
                gb[sl] = mb[0, sl] * 2 + c
            pltpu.async_copy(xr_hbm.at[gb], rows, sem).wait()

            def grp(g, inner):
                wg = wb[0, pl.ds(g * _LANES, _LANES)]
                for r16 in range(_LANES):
                    r = g * _LANES + r16
                    wsc = wg[r16]
                    for v in range(dh // _LANES):
                        sl = pl.ds(v * _LANES, _LANES)
                        rows[r, sl] = rows[r, sl] * wsc
                return inner

            lax.fori_loop(0, _CHUNK // _LANES, grp, None)
            pltpu.sync_copy(rows, agg.at[mb.at[1]], add=True)
            return carry

        lax.fori_loop(0, nch, chunk, None)
        plsc.subcore_barrier()
        pltpu.sync_copy(agg.at[pl.ds(rbase, rpt)], out_hbm.at[c, pl.ds(rbase, rpt)])
        if rem:
            @pl.when(s == _NS - 1)
            def _write_tail():
                pltpu.sync_copy(agg.at[pl.ds(tail, rem)], out_hbm.at[c, pl.ds(tail, rem)])

    return k(xr, meta, wr)


def _dense_relu(agg, W):
    _, n, dh = agg.shape
    d_out = W.shape[1]
    bm = 1000

    def body(a_ref, w_ref, o_ref):
        a = a_ref[...]
        w = w_ref[...]
        y = jnp.dot(a[0], w[:dh], preferred_element_type=jnp.float32)
        y = y + jnp.dot(a[1], w[dh:], preferred_element_type=jnp.float32)
        o_ref[...] = jnp.maximum(y, 0.0)

    return pl.pallas_call(
        body,
        grid=(n // bm,),
        in_specs=[
            pl.BlockSpec((2, bm, dh), lambda i: (0, i, 0)),
            pl.BlockSpec(W.shape, lambda i: (0, 0)),
        ],
        out_specs=pl.BlockSpec((bm, d_out), lambda i: (i, 0)),
        out_shape=jax.ShapeDtypeStruct((n, d_out), jnp.float32),
    )(agg, W)


def kernel(x, edge_index, edge_weight, W):
    n, d = x.shape
    e = edge_weight.shape[0]
    dh = d // 2
    src = edge_index[0].astype(jnp.int32)
    dst = edge_index[1].astype(jnp.int32)
    xr = x.reshape(2 * n, dh)
    nch = e // (_NS * _CHUNK)
    srcr = src.reshape(_NS, nch, _CHUNK)
    dstr = dst.reshape(_NS, nch, _CHUNK)
    meta = jnp.stack([srcr, dstr], axis=2)
    wr = edge_weight.reshape(_NS, nch, 1, _CHUNK)
    agg = _spmm(xr, meta, wr, n)
    return _dense_relu(agg, W)


# speedup vs baseline: 6.6687x; 2.1687x over previous
import functools

import jax
import jax.numpy as jnp
from jax import lax
from jax.experimental import pallas as pl
from jax.experimental.pallas import tpu as pltpu
from jax.experimental.pallas import tpu_sc as plsc

_NC = 2
_NS = 16
_LANES = 16
_CHUNK = 80
_NB = 3


def _spmm(xr, meta, wr, n_nodes):
    _, dh = xr.shape
    n = n_nodes
    nch = meta.shape[1]
    rpt = (n // _NS) // 8 * 8
    rem = n - _NS * rpt
    nsteady = (nch - 2) // _NB * _NB if nch > 2 else 0
    tail = list(range(nsteady, nch))

    mesh = plsc.VectorSubcoreMesh(
        core_axis_name="c", subcore_axis_name="s", num_cores=_NC, num_subcores=_NS
    )

    @functools.partial(
        pl.kernel,
        mesh=mesh,
        out_type=jax.ShapeDtypeStruct((_NC, n, dh), jnp.float32),
        scratch_types=[
            pltpu.VMEM((_NB, 2, _CHUNK), jnp.int32),
            pltpu.VMEM((_NB, 1, _CHUNK), jnp.float32),
            pltpu.VMEM((_NB, _CHUNK), jnp.int32),
            pltpu.VMEM((_NB, _CHUNK, dh), jnp.float32),
            pltpu.VMEM_SHARED((n, dh), jnp.float32),
            [pltpu.SemaphoreType.DMA] * _NB,
            [pltpu.SemaphoreType.DMA] * _NB,
            [pltpu.SemaphoreType.DMA] * _NB,
        ],
    )
    def k(xr_hbm, meta_hbm, w_hbm, out_hbm, mb, wb, gb, rows, agg,
          sem_m, sem_g, sem_s):
        c = lax.axis_index("c")
        s = lax.axis_index("s")
        rbase = pl.multiple_of(s * rpt, 8)
        tbase = _NS * rpt

        def issue_meta(j, b):
            pltpu.async_copy(meta_hbm.at[s, j], mb.at[b], sem_m[b])
            pltpu.async_copy(w_hbm.at[s, j], wb.at[b], sem_m[b])

        def wait_meta(j, b):
            pltpu.make_async_copy(meta_hbm.at[s, j], mb.at[b], sem_m[b]).wait()
            pltpu.make_async_copy(w_hbm.at[s, j], wb.at[b], sem_m[b]).wait()

        def prep_gather(b):
            for v in range(_CHUNK // _LANES):
                sl = pl.ds(v * _LANES, _LANES)
                gb[b, sl] = mb[b, 0, sl] * 2 + c
            pltpu.async_copy(xr_hbm.at[gb.at[b]], rows.at[b], sem_g[b])

        def wait_gather(b):
            pltpu.make_async_copy(xr_hbm.at[gb.at[b]], rows.at[b], sem_g[b]).wait()

        def scale(b):
            def grp(g, inner):
                wg = wb[b, 0, pl.ds(g * _LANES, _LANES)]
                for r16 in range(_LANES):
                    r = g * _LANES + r16
                    wsc = wg[r16]
                    for v in range(dh // _LANES):
                        sl = pl.ds(v * _LANES, _LANES)
                        rows[b, r, sl] = rows[b, r, sl] * wsc
                return inner

            lax.fori_loop(0, _CHUNK // _LANES, grp, None)

        def issue_scatter(b):
            pltpu.async_copy(rows.at[b], agg.at[mb.at[b, 1]], sem_s[b], add=True)

        def wait_scatter(b):
            pltpu.make_async_copy(rows.at[b], agg.at[mb.at[b, 1]], sem_s[b]).wait()

        def step(j, k_slot, guard_prev, do_fetch, do_next):
            b = k_slot % _NB
            bp = (k_slot + _NB - 1) % _NB
            bn = (k_slot + 1) % _NB
            if guard_prev:
                @pl.when(j >= 1)
                def _():
                    wait_scatter(bp)
            else:
                wait_scatter(bp)
            if do_fetch:
                issue_meta(j + 2, bp)
            if do_next:
                wait_meta(j + 1, bn)
                prep_gather(bn)
            wait_gather(b)
            scale(b)
            issue_scatter(b)

        def zrow(r, carry):
            for v in range(dh // _LANES):
                rows[0, r, pl.ds(v * _LANES, _LANES)] = jnp.zeros(
                    (_LANES,), jnp.float32
                )
            return carry

        issue_meta(0, 0)
        issue_meta(1, 1)
        lax.fori_loop(0, _CHUNK, zrow, None)
        nz_full = rpt // _CHUNK
        for kz in range(nz_full):
            pltpu.sync_copy(rows.at[0], agg.at[pl.ds(rbase + kz * _CHUNK, _CHUNK)])
        zrem = rpt - nz_full * _CHUNK
        if zrem:
            pltpu.sync_copy(
                rows.at[0, pl.ds(0, zrem)],
                agg.at[pl.ds(rbase + nz_full * _CHUNK, zrem)],
            )
        if rem:
            @pl.when(s == _NS - 1)
            def _zero_tail():
                pltpu.sync_copy(rows.at[0, pl.ds(0, rem)], agg.at[pl.ds(tbase, rem)])
        plsc.subcore_barrier()

        wait_meta(0, 0)
        prep_gather(0)

        def fbody(jj, carry):
            j0 = jj * _NB
            for k_slot in range(_NB):
                step(j0 + k_slot, k_slot, guard_prev=(k_slot == 0),
                     do_fetch=True, do_next=True)
            return carry

        lax.fori_loop(0, nsteady // _NB, fbody, None)
        for j in tail:
            step(j, j % _NB, guard_prev=(j == 0),
                 do_fetch=(j + 2 < nch), do_next=(j + 1 < nch))
        wait_scatter((nch - 1) % _NB)
        plsc.subcore_barrier()

        pltpu.sync_copy(agg.at[pl.ds(rbase, rpt)], out_hbm.at[c, pl.ds(rbase, rpt)])
        if rem:
            @pl.when(s == _NS - 1)
            def _write_tail():
                pltpu.sync_copy(agg.at[pl.ds(tbase, rem)], out_hbm.at[c, pl.ds(tbase, rem)])

    return k(xr, meta, wr)


def _dense_relu(agg, W):
    _, n, dh = agg.shape
    d_out = W.shape[1]
    bm = 1000

    def body(a_ref, w_ref, o_ref):
        a = a_ref[...]
        w = w_ref[...]
        y = jnp.dot(a[0], w[:dh], preferred_element_type=jnp.float32)
        y = y + jnp.dot(a[1], w[dh:], preferred_element_type=jnp.float32)
        o_ref[...] = jnp.maximum(y, 0.0)

    return pl.pallas_call(
        body,
        grid=(n // bm,),
        in_specs=[
            pl.BlockSpec((2, bm, dh), lambda i: (0, i, 0)),
            pl.BlockSpec(W.shape, lambda i: (0, 0)),
        ],
        out_specs=pl.BlockSpec((bm, d_out), lambda i: (i, 0)),
        out_shape=jax.ShapeDtypeStruct((n, d_out), jnp.float32),
    )(agg, W)


def kernel(x, edge_index, edge_weight, W):
    n, d = x.shape
    e = edge_weight.shape[0]
    dh = d // 2
    src = edge_index[0].astype(jnp.int32)
    dst = edge_index[1].astype(jnp.int32)
    xr = x.reshape(2 * n, dh)
    nch = e // (_NS * _CHUNK)
    srcr = src.reshape(_NS, nch, _CHUNK)
    dstr = dst.reshape(_NS, nch, _CHUNK)
    meta = jnp.stack([srcr, dstr], axis=2)
    wr = edge_weight.reshape(_NS, nch, 1, _CHUNK)
    agg = _spmm(xr, meta, wr, n)
    return _dense_relu(agg, W)


# speedup vs baseline: 7.2934x; 1.0937x over previous
import functools

import jax
import jax.numpy as jnp
from jax import lax
from jax.experimental import pallas as pl
from jax.experimental.pallas import tpu as pltpu
from jax.experimental.pallas import tpu_sc as plsc

_NC = 2
_NS = 16
_LANES = 16
_CHUNK = 80
_NB = 3


def _spmm(xr, meta, wr, n_nodes):
    _, dh = xr.shape
    n = n_nodes
    nch = meta.shape[1]
    rpt = (n // _NS) // 8 * 8
    rem = n - _NS * rpt
    nsteady = (nch - 2) // _NB * _NB if nch > 2 else 0
    tail = list(range(nsteady, nch))

    mesh = plsc.VectorSubcoreMesh(
        core_axis_name="c", subcore_axis_name="s", num_cores=_NC, num_subcores=_NS
    )

    @functools.partial(
        pl.kernel,
        mesh=mesh,
        out_type=jax.ShapeDtypeStruct((_NC, n, dh), jnp.float32),
        scratch_types=[
            pltpu.VMEM((_NB, 2, _CHUNK), jnp.int32),
            pltpu.VMEM((_NB, 1, _CHUNK), jnp.float32),
            pltpu.VMEM((_NB, _CHUNK), jnp.int32),
            pltpu.VMEM((_NB, _CHUNK), jnp.int32),
            pltpu.VMEM((_NB, _CHUNK, dh), jnp.float32),
            pltpu.VMEM_SHARED((n, dh), jnp.float32),
            [pltpu.SemaphoreType.DMA] * _NB,
            [pltpu.SemaphoreType.DMA] * _NB,
            [pltpu.SemaphoreType.DMA] * _NB,
        ],
    )
    def k(xr_hbm, meta_hbm, w_hbm, out_hbm, mb, wb, gb, db, rows, agg,
          sem_m, sem_g, sem_s):
        c = lax.axis_index("c")
        s = lax.axis_index("s")
        rbase = pl.multiple_of(s * rpt, 8)
        tbase = _NS * rpt

        def issue_meta(j, b):
            pltpu.async_copy(meta_hbm.at[s, j], mb.at[b], sem_m[b])
            pltpu.async_copy(w_hbm.at[s, j], wb.at[b], sem_m[b])

        def wait_meta(j, b):
            pltpu.make_async_copy(meta_hbm.at[s, j], mb.at[b], sem_m[b]).wait()
            pltpu.make_async_copy(w_hbm.at[s, j], wb.at[b], sem_m[b]).wait()

        def prep_gather(b):
            for v in range(_CHUNK // _LANES):
                sl = pl.ds(v * _LANES, _LANES)
                gb[b, sl] = mb[b, 0, sl] * 2 + c
                db[b, sl] = mb[b, 1, sl]
            pltpu.async_copy(xr_hbm.at[gb.at[b]], rows.at[b], sem_g[b])

        def wait_gather(b):
            pltpu.make_async_copy(xr_hbm.at[gb.at[b]], rows.at[b], sem_g[b]).wait()

        def scale(b):
            def grp(g, inner):
                wg = wb[b, 0, pl.ds(g * _LANES, _LANES)]
                for r16 in range(_LANES):
                    r = g * _LANES + r16
                    wsc = wg[r16]
                    for v in range(dh // _LANES):
                        sl = pl.ds(v * _LANES, _LANES)
                        rows[b, r, sl] = rows[b, r, sl] * wsc
                return inner

            lax.fori_loop(0, _CHUNK // _LANES, grp, None)

        def issue_scatter(b):
            pltpu.async_copy(rows.at[b], agg.at[db.at[b]], sem_s[b], add=True)

        def wait_scatter(b):
            pltpu.make_async_copy(rows.at[b], agg.at[db.at[b]], sem_s[b]).wait()

        def step(j, k_slot, guard_prev, do_fetch, do_next):
            b = k_slot % _NB
            bp = (k_slot + _NB - 1) % _NB
            bn = (k_slot + 1) % _NB
            if guard_prev:
                @pl.when(j >= 2)
                def _():
                    wait_scatter(bn)
            else:
                wait_scatter(bn)
            if do_next:
                wait_meta(j + 1, bn)
                prep_gather(bn)
            if do_fetch:
                issue_meta(j + 2, bp)
            wait_gather(b)
            scale(b)
            issue_scatter(b)

        def zrow(r, carry):
            for v in range(dh // _LANES):
                rows[0, r, pl.ds(v * _LANES, _LANES)] = jnp.zeros(
                    (_LANES,), jnp.float32
                )
            return carry

        issue_meta(0, 0)
        issue_meta(1, 1)
        lax.fori_loop(0, _CHUNK, zrow, None)
        nz_full = rpt // _CHUNK
        for kz in range(nz_full):
            pltpu.sync_copy(rows.at[0], agg.at[pl.ds(rbase + kz * _CHUNK, _CHUNK)])
        zrem = rpt - nz_full * _CHUNK
        if zrem:
            pltpu.sync_copy(
                rows.at[0, pl.ds(0, zrem)],
                agg.at[pl.ds(rbase + nz_full * _CHUNK, zrem)],
            )
        if rem:
            @pl.when(s == _NS - 1)
            def _zero_tail():
                pltpu.sync_copy(rows.at[0, pl.ds(0, rem)], agg.at[pl.ds(tbase, rem)])
        plsc.subcore_barrier()

        wait_meta(0, 0)
        prep_gather(0)

        def fbody(jj, carry):
            j0 = jj * _NB
            for k_slot in range(_NB):
                step(j0 + k_slot, k_slot, guard_prev=(k_slot <= 1),
                     do_fetch=True, do_next=True)
            return carry

        lax.fori_loop(0, nsteady // _NB, fbody, None)
        for j in tail:
            step(j, j % _NB, guard_prev=(j < 2),
                 do_fetch=(j + 2 < nch), do_next=(j + 1 < nch))
        if nch >= 2:
            wait_scatter((nch - 2) % _NB)
        wait_scatter((nch - 1) % _NB)
        plsc.subcore_barrier()

        pltpu.sync_copy(agg.at[pl.ds(rbase, rpt)], out_hbm.at[c, pl.ds(rbase, rpt)])
        if rem:
            @pl.when(s == _NS - 1)
            def _write_tail():
                pltpu.sync_copy(agg.at[pl.ds(tbase, rem)], out_hbm.at[c, pl.ds(tbase, rem)])

    return k(xr, meta, wr)


def _dense_relu(agg, W):
    _, n, dh = agg.shape
    d_out = W.shape[1]
    bm = 1000

    def body(a_ref, w_ref, o_ref):
        a = a_ref[...]
        w = w_ref[...]
        y = jnp.dot(a[0], w[:dh], preferred_element_type=jnp.float32)
        y = y + jnp.dot(a[1], w[dh:], preferred_element_type=jnp.float32)
        o_ref[...] = jnp.maximum(y, 0.0)

    return pl.pallas_call(
        body,
        grid=(n // bm,),
        in_specs=[
            pl.BlockSpec((2, bm, dh), lambda i: (0, i, 0)),
            pl.BlockSpec(W.shape, lambda i: (0, 0)),
        ],
        out_specs=pl.BlockSpec((bm, d_out), lambda i: (i, 0)),
        out_shape=jax.ShapeDtypeStruct((n, d_out), jnp.float32),
    )(agg, W)


def kernel(x, edge_index, edge_weight, W):
    n, d = x.shape
    e = edge_weight.shape[0]
    dh = d // 2
    src = edge_index[0].astype(jnp.int32)
    dst = edge_index[1].astype(jnp.int32)
    xr = x.reshape(2 * n, dh)
    nch = e // (_NS * _CHUNK)
    srcr = src.reshape(_NS, nch, _CHUNK)
    dstr = dst.reshape(_NS, nch, _CHUNK)
    meta = jnp.stack([srcr, dstr], axis=2)
    wr = edge_weight.reshape(_NS, nch, 1, _CHUNK)
    agg = _spmm(xr, meta, wr, n)
    return _dense_relu(agg, W)
